# Initial kernel scaffold; baseline (speedup 1.0000x reference)
import jax
import jax.numpy as jnp
from jax.experimental import pallas as pl


def kernel(points1, points2, N_out, k, t, W1, b1, ga1, be1, W2, b2, ga2, be2, W3, b3, ga3, be3):
    raise NotImplementedError("write your pallas kernel here")


# speedup vs baseline: 17.4439x; 17.4439x over previous
import functools

import jax
import jax.numpy as jnp
from jax import lax
from jax.experimental import pallas as pl
from jax.experimental.pallas import tpu as pltpu
from jax.experimental.pallas import tpu_sc as plsc

EPS_BN = 0.001
B = 2
N_IN = 4096
N_S = 1024
K_S = 32
M_ROWS = B * N_S * K_S


def _rsum(a):
    return jnp.sum(jnp.sum(a, axis=2, keepdims=True), axis=1, keepdims=True)


def _rmax(a):
    return jnp.max(jnp.max(a, axis=2, keepdims=True), axis=1, keepdims=True)


def _rmin(a):
    return jnp.min(jnp.min(a, axis=2, keepdims=True), axis=1, keepdims=True)


def _fps(xs, ys, zs, *, interpret=False):
    def body(x_ref, y_ref, z_ref, o_ref):
        X = x_ref[...]
        Y = y_ref[...]
        Z = z_ref[...]
        r = lax.broadcasted_iota(jnp.int32, (4, 32, 128), 1)
        c = lax.broadcasted_iota(jnp.int32, (4, 32, 128), 2)
        flat = r * 128 + c
        lane3 = lax.broadcasted_iota(jnp.int32, (4, 1, N_S), 2)

        def step(i, st):
            dist, far, idxs = st
            idxs = jnp.where(lane3 == i, far, idxs)
            oh = flat == far
            cx = _rsum(jnp.where(oh, X, 0.0))
            cy = _rsum(jnp.where(oh, Y, 0.0))
            cz = _rsum(jnp.where(oh, Z, 0.0))
            dx = X - cx
            dy = Y - cy
            dz = Z - cz
            d = dx * dx + dy * dy + dz * dz
            dist = jnp.minimum(dist, d)
            m = _rmax(dist)
            far = _rmin(jnp.where(dist == m, flat, N_IN))
            return dist, far, idxs

        dist0 = jnp.full((4, 32, 128), 1e10, jnp.float32)
        far0 = jnp.zeros((4, 1, 1), jnp.int32)
        idxs0 = jnp.zeros((4, 1, N_S), jnp.int32)
        _, _, idxs = lax.fori_loop(0, N_S, step, (dist0, far0, idxs0))
        o_ref[...] = idxs

    return pl.pallas_call(
        body,
        out_shape=jax.ShapeDtypeStruct((4, 1, N_S), jnp.int32),
        interpret=interpret,
    )(xs, ys, zs)


def _knn(nvalid, qT, rs, *, interpret=False):
    def body(nv_ref, q_ref, r_ref, o_ref):
        pid = pl.program_id(0)
        nv = nv_ref[pid]
        qx = q_ref[0, :, 0:1]
        qy = q_ref[0, :, 1:2]
        qz = q_ref[0, :, 2:3]
        rx = r_ref[0, 0:1, :]
        ry = r_ref[0, 1:2, :]
        rz = r_ref[0, 2:3, :]
        dx = qx - rx
        dy = qy - ry
        dz = qz - rz
        d2 = dx * dx + dy * dy + dz * dz
        lane = lax.broadcasted_iota(jnp.int32, (1, N_S), 1)
        d2 = jnp.where(lane >= nv, jnp.inf, d2)
        for j in range(K_S):
            m = jnp.min(d2, axis=1, keepdims=True)
            hit = d2 == m
            sel = jnp.min(jnp.where(hit, lane, N_S * 2), axis=1, keepdims=True)
            o_ref[0, :, j:j + 1] = sel
            if j + 1 < K_S:
                d2 = jnp.where(lane == sel, jnp.inf, d2)

    return pl.pallas_call(
        body,
        grid=(4,),
        in_specs=[
            pl.BlockSpec(memory_space=pltpu.SMEM),
            pl.BlockSpec((1, N_S, 3), lambda i: (i, 0, 0)),
            pl.BlockSpec((1, 3, N_S), lambda i: (i, 0, 0)),
        ],
        out_specs=pl.BlockSpec((1, N_S, K_S), lambda i: (i, 0, 0)),
        out_shape=jax.ShapeDtypeStruct((4, N_S, K_S), jnp.int32),
        interpret=interpret,
    )(nvalid, qT, rs)


def _gather_rows(table, idx3d):
    info = plsc.get_sparse_core_info()
    nc, ns = info.num_cores, info.num_subcores
    nw = nc * ns
    bpw = M_ROWS // nw
    nchunk = bpw // 128

    mesh = plsc.VectorSubcoreMesh(core_axis_name="c", subcore_axis_name="s")

    @functools.partial(
        pl.kernel,
        mesh=mesh,
        compiler_params=pltpu.CompilerParams(use_tc_tiling_on_sc=False),
        out_type=jax.ShapeDtypeStruct((M_ROWS, 16), jnp.float32),
        scratch_types=[
            pltpu.VMEM((nchunk, 128), jnp.int32),
            pltpu.VMEM((bpw, 16), jnp.float32),
            pltpu.SemaphoreType.DMA,
        ],
    )
    def k(table_hbm, idx_hbm, out_hbm, idx_v, rows_v, sem):
        wid = lax.axis_index("s") * nc + lax.axis_index("c")
        base = wid * bpw
        pltpu.sync_copy(idx_hbm.at[wid], idx_v)
        descs = [
            pltpu.async_copy(
                table_hbm.at[idx_v.at[j]],
                rows_v.at[pl.ds(j * 128, 128)],
                sem,
            )
            for j in range(nchunk)
        ]
        for d in descs:
            d.wait()
        pltpu.sync_copy(rows_v, out_hbm.at[pl.ds(base, bpw)])

    return k(table, idx3d)


def _mlp(nnT, qT, W1, W2, W3, P1, P2, P3, *, interpret=False):
    CH = 8192
    NCH = M_ROWS // CH
    M = float(M_ROWS)

    def body(nn_ref, q_ref, w1_ref, w2_ref, w3_ref, p1_ref, p2_ref, p3_ref,
             s_ref, a1_ref, a2_ref):
        w1 = w1_ref[...]
        w2 = w2_ref[...]
        w3 = w3_ref[...]
        b1, g1, e1 = p1_ref[:, 0:1], p1_ref[:, 1:2], p1_ref[:, 2:3]
        b2, g2, e2 = p2_ref[:, 0:1], p2_ref[:, 1:2], p2_ref[:, 2:3]
        b3, g3, e3 = p3_ref[:, 0:1], p3_ref[:, 1:2], p3_ref[:, 2:3]
        dn = (((1,), (0,)), ((), ()))

        s1 = jnp.zeros((64, 1), jnp.float32)
        for c in range(NCH):
            sl = slice(c * CH, (c + 1) * CH)
            nn = nn_ref[:, sl]
            q = q_ref[:, sl]
            resi = nn - q
            rr = resi * resi
            d2 = rr[0:1, :] + rr[1:2, :] + rr[2:3, :]
            X = jnp.concatenate([resi, jnp.sqrt(d2)], axis=0)
            y = lax.dot_general(w1, X, dn,
                                preferred_element_type=jnp.float32) + b1
            a1_ref[:, sl] = y
            s1 = s1 + jnp.sum(y, axis=1, keepdims=True)
        m1 = s1 / M
        v1 = jnp.zeros((64, 1), jnp.float32)
        for c in range(NCH):
            sl = slice(c * CH, (c + 1) * CH)
            yc = a1_ref[:, sl] - m1
            v1 = v1 + jnp.sum(yc * yc, axis=1, keepdims=True)
        den1 = jnp.sqrt(v1 / M + EPS_BN)

        s2 = jnp.zeros((64, 1), jnp.float32)
        for c in range(NCH):
            sl = slice(c * CH, (c + 1) * CH)
            a = jnp.maximum((a1_ref[:, sl] - m1) / den1 * g1 + e1, 0.0)
            a1_ref[:, sl] = a
            y = lax.dot_general(w2, a, dn,
                                preferred_element_type=jnp.float32) + b2
            a2_ref[:, sl] = y
            s2 = s2 + jnp.sum(y, axis=1, keepdims=True)
        m2 = s2 / M
        v2 = jnp.zeros((64, 1), jnp.float32)
        for c in range(NCH):
            sl = slice(c * CH, (c + 1) * CH)
            yc = a2_ref[:, sl] - m2
            v2 = v2 + jnp.sum(yc * yc, axis=1, keepdims=True)
        den2 = jnp.sqrt(v2 / M + EPS_BN)

        s3 = jnp.zeros((128, 1), jnp.float32)
        for c in range(NCH):
            sl = slice(c * CH, (c + 1) * CH)
            a = jnp.maximum((a2_ref[:, sl] - m2) / den2 * g2 + e2, 0.0)
            a2_ref[:, sl] = a
            y = lax.dot_general(w3, a, dn,
                                preferred_element_type=jnp.float32) + b3
            s3 = s3 + jnp.sum(y, axis=1, keepdims=True)
        m3 = s3 / M
        v3 = jnp.zeros((128, 1), jnp.float32)
        for c in range(NCH):
            sl = slice(c * CH, (c + 1) * CH)
            y = lax.dot_general(w3, a2_ref[:, sl], dn,
                                preferred_element_type=jnp.float32) + b3
            yc = y - m3
            v3 = v3 + jnp.sum(yc * yc, axis=1, keepdims=True)
        den3 = jnp.sqrt(v3 / M + EPS_BN)
        for c in range(NCH):
            sl = slice(c * CH, (c + 1) * CH)
            y = lax.dot_general(w3, a2_ref[:, sl], dn,
                                preferred_element_type=jnp.float32) + b3
            a = jnp.maximum((y - m3) / den3 * g3 + e3, 0.0)
            s_ref[0:1, sl] = jnp.max(a, axis=0, keepdims=True)

    return pl.pallas_call(
        body,
        out_shape=jax.ShapeDtypeStruct((1, M_ROWS), jnp.float32),
        scratch_shapes=[
            pltpu.VMEM((64, M_ROWS), jnp.float32),
            pltpu.VMEM((64, M_ROWS), jnp.float32),
        ],
        interpret=interpret,
    )(nnT, qT, W1, W2, W3, P1, P2, P3)


def _smax(s2d, gx, gy, gz, *, interpret=False):
    def body(s_ref, x_ref, y_ref, z_ref, o_ref):
        s = s_ref[...]
        m = jnp.max(s, axis=1, keepdims=True)
        e = jnp.exp(s - m)
        w = e / jnp.sum(e, axis=1, keepdims=True)
        o_ref[:, 0:1] = jnp.sum(w * x_ref[...], axis=1, keepdims=True)
        o_ref[:, 1:2] = jnp.sum(w * y_ref[...], axis=1, keepdims=True)
        o_ref[:, 2:3] = jnp.sum(w * z_ref[...], axis=1, keepdims=True)

    return pl.pallas_call(
        body,
        out_shape=jax.ShapeDtypeStruct((B * N_S, 3), jnp.float32),
        interpret=interpret,
    )(s2d, gx, gy, gz)


def kernel(points1, points2, N_out, k, t,
           W1, b1, ga1, be1, W2, b2, ga2, be2, W3, b3, ga3, be3):
    allpts = jnp.stack([points1, points2], axis=1).reshape(2 * B, 3, N_IN)
    xs = allpts[:, 0, :].reshape(2 * B, 32, 128)
    ys = allpts[:, 1, :].reshape(2 * B, 32, 128)
    zs = allpts[:, 2, :].reshape(2 * B, 32, 128)
    fps_idx = _fps(xs, ys, zs).reshape(2 * B, N_S)
    samp = jnp.take_along_axis(allpts, fps_idx[:, None, :], axis=2)
    p1full = samp[0::2]
    p2full = samp[1::2]

    N2 = jnp.floor(N_out * t).astype(jnp.int32)
    N1 = (N_out - N2).astype(jnp.int32)
    cols = jnp.arange(N_S, dtype=jnp.int32)
    shift_n = jnp.clip(cols[None, :] - N1[:, None], 0, N_S - 1)
    p2sh = jnp.take_along_axis(
        p2full, jnp.broadcast_to(shift_n[:, None, :], (B, 3, N_S)), axis=2)
    newp = jnp.where((cols[None, None, :] < N1[:, None, None]), p1full, p2sh)

    qs = jnp.repeat(newp, 2, axis=0)
    qsT = qs.transpose(0, 2, 1)
    nvalid = jnp.stack([N1, N2], axis=1).reshape(2 * B)
    knn_idx = _knn(nvalid, qsT, samp)

    k2 = jnp.floor(k * t).astype(jnp.int32)
    k1 = (k - k2).astype(jnp.int32)
    kc = jnp.arange(K_S, dtype=jnp.int32)
    idx1 = knn_idx[0::2]
    idx2 = knn_idx[1::2]
    shift_k = jnp.clip(kc[None, :] - k1[:, None], 0, K_S - 1)
    idx2s = jnp.take_along_axis(
        idx2, jnp.broadcast_to(shift_k[:, None, :], (B, N_S, K_S)), axis=2)
    idxc = jnp.where(kc[None, None, :] < k1[:, None, None], idx1, idx2s + N_S)
    idxg = idxc + (jnp.arange(B, dtype=jnp.int32) * 2 * N_S)[:, None, None]
    idx3d = idxg.reshape(32, 16, 128).astype(jnp.int32)

    table = samp.transpose(0, 2, 1).reshape(2 * B * N_S, 3)
    tablep = jnp.concatenate(
        [table, jnp.zeros((2 * B * N_S, 13), jnp.float32)], axis=1)
    gath = _gather_rows(tablep, idx3d)

    nnT = gath[:, :3].T
    qT = jnp.broadcast_to(
        newp[:, :, :, None], (B, 3, N_S, K_S)).transpose(1, 0, 2, 3).reshape(
            3, M_ROWS)
    P1 = jnp.stack([b1, ga1, be1], axis=1)
    P2 = jnp.stack([b2, ga2, be2], axis=1)
    P3 = jnp.stack([b3, ga3, be3], axis=1)
    s = _mlp(nnT, qT, W1, W2, W3, P1, P2, P3)

    s2d = s.reshape(B * N_S, K_S)
    gx = gath[:, 0].reshape(B * N_S, K_S)
    gy = gath[:, 1].reshape(B * N_S, K_S)
    gz = gath[:, 2].reshape(B * N_S, K_S)
    res = _smax(s2d, gx, gy, gz)
    return res.reshape(B, N_S, 3).transpose(0, 2, 1)
